# baseline (speedup 1.0000x reference)
import functools

import jax
import jax.numpy as jnp
from jax import lax
from jax.experimental import pallas as pl
from jax.experimental.pallas import tpu as pltpu
from jax.experimental.pallas import tpu_sc as plsc

_NC = 2
_NS = 16
_NW = _NC * _NS


def _make_gather(B, D, chunk):
    n_chunks_total = B // chunk
    n_chunks_w = n_chunks_total // _NW
    bpw = n_chunks_w * chunk

    def body(tokens_hbm, table_hbm, out_hbm, idx_v, rows_v, sem):
        wid = lax.axis_index("s") * _NC + lax.axis_index("c")
        base = wid * bpw

        def step(i, carry):
            off = base + i * chunk
            pltpu.sync_copy(tokens_hbm.at[pl.ds(off, chunk)], idx_v)
            pltpu.async_copy(table_hbm.at[idx_v], rows_v, sem).wait()
            pltpu.sync_copy(rows_v, out_hbm.at[pl.ds(off, chunk)])
            return carry

        lax.fori_loop(0, n_chunks_w, step, 0)

    mesh = plsc.VectorSubcoreMesh(core_axis_name="c", subcore_axis_name="s")
    return pl.kernel(
        body,
        out_type=jax.ShapeDtypeStruct((B, D), jnp.float32),
        mesh=mesh,
        scratch_types=[
            pltpu.VMEM((chunk,), jnp.int32),
            pltpu.VMEM((chunk, D), jnp.float32),
            pltpu.SemaphoreType.DMA,
        ],
        compiler_params=pltpu.CompilerParams(use_tc_tiling_on_sc=False),
    )


def kernel(tokens, table):
    n, m = tokens.shape
    vocab, dim = table.shape
    B = n * m
    flat = tokens.reshape(B).astype(jnp.int32)
    out = _make_gather(B, dim, 1024)(flat, table)
    return out.reshape(n, m, dim)


# speedup vs baseline: 1.0078x; 1.0078x over previous
import functools

import jax
import jax.numpy as jnp
from jax import lax
from jax.experimental import pallas as pl
from jax.experimental.pallas import tpu as pltpu
from jax.experimental.pallas import tpu_sc as plsc

_NC = 2
_NS = 16
_NW = _NC * _NS


def _make_gather(B, D, chunk, nbuf):
    bpw = B // _NW
    n_groups = bpw // (chunk * nbuf)
    assert bpw % (chunk * nbuf) == 0

    def body(tokens_hbm, table_hbm, out_hbm, idx_v, rows_v, *sems):
        sem_g = sems[:nbuf]
        sem_s = sems[nbuf:]
        wid = lax.axis_index("s") * _NC + lax.axis_index("c")
        base = wid * bpw

        pltpu.sync_copy(tokens_hbm.at[pl.ds(base, bpw)], idx_v)

        def group(g, carry):
            gathers = []
            for b in range(nbuf):
                off = (g * nbuf + b) * chunk

                @pl.when(g > 0)
                def _wait_prev():
                    pltpu.make_async_copy(
                        rows_v.at[b], out_hbm.at[pl.ds(base, chunk)], sem_s[b]
                    ).wait()

                gathers.append(
                    pltpu.async_copy(
                        table_hbm.at[idx_v.at[pl.ds(off, chunk)]],
                        rows_v.at[b],
                        sem_g[b],
                    )
                )
            for b in range(nbuf):
                off = (g * nbuf + b) * chunk
                gathers[b].wait()
                pltpu.async_copy(
                    rows_v.at[b], out_hbm.at[pl.ds(base + off, chunk)], sem_s[b]
                )
            return carry

        lax.fori_loop(0, n_groups, group, 0)

        for b in range(nbuf):
            pltpu.make_async_copy(
                rows_v.at[b], out_hbm.at[pl.ds(base, chunk)], sem_s[b]
            ).wait()

    mesh = plsc.VectorSubcoreMesh(core_axis_name="c", subcore_axis_name="s")
    return pl.kernel(
        body,
        out_type=jax.ShapeDtypeStruct((B, D), jnp.float32),
        mesh=mesh,
        scratch_types=[
            pltpu.VMEM((bpw,), jnp.int32),
            pltpu.VMEM((nbuf, chunk, D), jnp.float32),
        ]
        + [pltpu.SemaphoreType.DMA] * (2 * nbuf),
        compiler_params=pltpu.CompilerParams(use_tc_tiling_on_sc=False),
    )


def kernel(tokens, table):
    n, m = tokens.shape
    vocab, dim = table.shape
    B = n * m
    flat = tokens.reshape(B).astype(jnp.int32)
    out = _make_gather(B, dim, 512, 2)(flat, table)
    return out.reshape(n, m, dim)
